# Initial kernel scaffold; baseline (speedup 1.0000x reference)
import jax
import jax.numpy as jnp
from jax.experimental import pallas as pl


def kernel(features, neigh_idx, num_sample):
    raise NotImplementedError("write your pallas kernel here")


# speedup vs baseline: 3.5736x; 3.5736x over previous
import functools

import jax
import jax.numpy as jnp
from jax import lax
from jax.experimental import pallas as pl
from jax.experimental.pallas import tpu as pltpu
from jax.experimental.pallas import tpu_sc as plsc

N_NODES = 100000
D_FEAT = 128
BATCH = 50000
LANES = 16

NC, NS = 2, 16
NW = NC * NS


def _mean_agg_kernel(nchunks, nb, s, features_hbm, idx_hbm, out_hbm,
                     idx_v, rows_v, out_v, sem):
    wid = lax.axis_index("s") * NC + lax.axis_index("c")
    pltpu.sync_copy(idx_hbm.at[wid], idx_v)

    inv_s = jnp.float32(1.0 / s)

    def body(c, carry):
        pltpu.async_copy(features_hbm.at[idx_v.at[c]], rows_v, sem).wait()
        for r in range(nb):
            for d in range(D_FEAT // LANES):
                acc = rows_v[r * s, pl.ds(d * LANES, LANES)]
                for j in range(1, s):
                    acc = acc + rows_v[r * s + j, pl.ds(d * LANES, LANES)]
                out_v[r, pl.ds(d * LANES, LANES)] = acc * inv_s
        pltpu.sync_copy(out_v, out_hbm.at[pl.ds(wid * nchunks * nb + c * nb, nb)])
        return carry

    lax.fori_loop(0, nchunks, body, 0)


def kernel(features, neigh_idx, num_sample):
    del num_sample
    b, s = neigh_idx.shape
    nb = (128 // s) // 8 * 8
    assert nb >= 8
    rows_per_worker_chunks = -(-b // (NW * nb))
    nchunks = rows_per_worker_chunks
    b_pad = NW * nchunks * nb

    idx = jnp.zeros((b_pad, s), jnp.int32).at[:b].set(neigh_idx)
    idx = idx.reshape(NW, nchunks, nb * s)

    mesh = plsc.VectorSubcoreMesh(core_axis_name="c", subcore_axis_name="s",
                                  num_cores=NC, num_subcores=NS)
    out = pl.kernel(
        functools.partial(_mean_agg_kernel, nchunks, nb, s),
        out_type=jax.ShapeDtypeStruct((b_pad, D_FEAT), jnp.float32),
        mesh=mesh,
        scratch_types=[
            pltpu.VMEM((nchunks, nb * s), jnp.int32),
            pltpu.VMEM((nb * s, D_FEAT), jnp.float32),
            pltpu.VMEM((nb, D_FEAT), jnp.float32),
            pltpu.SemaphoreType.DMA,
        ],
    )(features, idx)
    return out[:b]


# speedup vs baseline: 4.6420x; 1.2990x over previous
import functools

import jax
import jax.numpy as jnp
from jax import lax
from jax.experimental import pallas as pl
from jax.experimental.pallas import tpu as pltpu
from jax.experimental.pallas import tpu_sc as plsc

N_NODES = 100000
D_FEAT = 128
BATCH = 50000
LANES = 16

NC, NS = 2, 16
NW = NC * NS


def _mean_agg_kernel(nchunks, nb, s, features_hbm, idx_hbm, out_hbm,
                     idx_v, rows_v, out_v, g0, g1, o0, o1):
    wid = lax.axis_index("s") * NC + lax.axis_index("c")
    pltpu.sync_copy(idx_hbm.at[wid], idx_v)

    inv_s = jnp.float32(1.0 / s)
    gsems = (g0, g1)
    osems = (o0, o1)
    out_base = wid * nchunks * nb

    for par in range(2):
        pltpu.async_copy(features_hbm.at[idx_v.at[par]], rows_v.at[par],
                         gsems[par])

    def body(i, carry):
        c2 = i * 2
        for par in range(2):
            c = c2 + par
            pltpu.make_async_copy(features_hbm.at[idx_v.at[c]],
                                  rows_v.at[par], gsems[par]).wait()
            @pl.when(c >= 2)
            def _():
                pltpu.make_async_copy(out_v.at[par],
                                      out_hbm.at[pl.ds(0, nb)],
                                      osems[par]).wait()
            for r in range(nb):
                for d in range(D_FEAT // LANES):
                    acc = rows_v[par, r * s, pl.ds(d * LANES, LANES)]
                    for j in range(1, s):
                        acc = acc + rows_v[par, r * s + j,
                                           pl.ds(d * LANES, LANES)]
                    out_v[par, r, pl.ds(d * LANES, LANES)] = acc * inv_s
            @pl.when(c + 2 < nchunks)
            def _():
                pltpu.async_copy(features_hbm.at[idx_v.at[c + 2]],
                                 rows_v.at[par], gsems[par])
            pltpu.async_copy(out_v.at[par],
                             out_hbm.at[pl.ds(out_base + c * nb, nb)],
                             osems[par])
        return carry

    lax.fori_loop(0, nchunks // 2, body, 0)

    for par in range(2):
        pltpu.make_async_copy(out_v.at[par], out_hbm.at[pl.ds(0, nb)],
                              osems[par]).wait()


def kernel(features, neigh_idx, num_sample):
    del num_sample
    b, s = neigh_idx.shape
    nb = (128 // s) // 8 * 8
    assert nb >= 8
    nchunks = -(-b // (NW * nb))
    nchunks += nchunks % 2
    b_pad = NW * nchunks * nb

    idx = jnp.zeros((b_pad, s), jnp.int32).at[:b].set(neigh_idx)
    idx = idx.reshape(NW, nchunks, nb * s)

    mesh = plsc.VectorSubcoreMesh(core_axis_name="c", subcore_axis_name="s",
                                  num_cores=NC, num_subcores=NS)
    out = pl.kernel(
        functools.partial(_mean_agg_kernel, nchunks, nb, s),
        out_type=jax.ShapeDtypeStruct((b_pad, D_FEAT), jnp.float32),
        mesh=mesh,
        scratch_types=[
            pltpu.VMEM((nchunks, nb * s), jnp.int32),
            pltpu.VMEM((2, nb * s, D_FEAT), jnp.float32),
            pltpu.VMEM((2, nb, D_FEAT), jnp.float32),
            pltpu.SemaphoreType.DMA,
            pltpu.SemaphoreType.DMA,
            pltpu.SemaphoreType.DMA,
            pltpu.SemaphoreType.DMA,
        ],
    )(features, idx)
    return out[:b]
